# baseline (speedup 1.0000x reference)
import functools

import jax
import jax.numpy as jnp
from jax import lax
from jax.experimental import pallas as pl
from jax.experimental.pallas import tpu as pltpu
from jax.experimental.pallas import tpu_sc as plsc

B, L, D = 4096, 200, 64
NC, NS = 2, 16
NW = NC * NS
TCB = 128
NTC = B // TCB
NTR = D // 8
NBUF = 4
PITCH = 133


def _make_kernel():
  mesh = plsc.VectorSubcoreMesh(core_axis_name="c", subcore_axis_name="s")

  @functools.partial(
      pl.kernel,
      mesh=mesh,
      compiler_params=pltpu.CompilerParams(use_tc_tiling_on_sc=False,
                                           needs_layout_passes=False),
      out_type=jax.ShapeDtypeStruct((L, NTR, NTC, 8, TCB), jnp.float32),
      scratch_types=(
          [pltpu.VMEM((L, TCB), jnp.int32),
           pltpu.VMEM((L, D), jnp.float32)]
          + [pltpu.VMEM((TCB, D), jnp.float32) for _ in range(NBUF)]
          + [pltpu.VMEM((D, PITCH), jnp.float32) for _ in range(NBUF)]
          + [pltpu.SemaphoreType.DMA for _ in range(2 * NBUF)]
      ),
  )
  def emb_kernel(seq_hbm, table_hbm, pe_hbm, out_hbm,
                 idx_all, pe_v, rows0, rows1, rows2, rows3,
                 tr0, tr1, tr2, tr3,
                 gs0, gs1, gs2, gs3, ws0, ws1, ws2, ws3):
    wid = lax.axis_index("s") * NC + lax.axis_index("c")
    b0 = pl.multiple_of(wid * TCB, TCB)

    pltpu.sync_copy(seq_hbm.at[:, pl.ds(b0, TCB)], idx_all)
    pltpu.sync_copy(pe_hbm.at[pl.ds(0, L)], pe_v)

    def fire_gather(l, rows_v, sem):
      pltpu.async_copy(table_hbm.at[idx_all.at[l]], rows_v, sem)

    def wait_gather(rows_v, sem):
      pltpu.make_async_copy(table_hbm.at[pl.ds(0, TCB)], rows_v, sem).wait()

    def fire_write(l, trans_v, rows_v, sem):
      for tr in range(NTR):
        pltpu.async_copy(trans_v.at[pl.ds(tr * 8, 8), pl.ds(0, TCB)],
                         out_hbm.at[l, tr, wid], sem)

    def wait_write(rows_v, sem):
      pltpu.make_async_copy(table_hbm.at[pl.ds(0, TCB)], rows_v, sem).wait()

    iota16 = lax.iota(jnp.int32, 16)
    zeros16 = jnp.zeros((16,), jnp.int32)
    dvecs = [iota16 + (dg * 16) for dg in range(D // 16)]

    def transpose_add(l, rows_v, trans_v):
      pev = [pe_v[l, pl.ds(dg * 16, 16)] for dg in range(D // 16)]

      @plsc.parallel_loop(0, TCB, step=2, unroll=4)
      def rbody(r0):
        for ru in range(2):
          r = r0 + ru
          bvec = zeros16 + r
          for dg in range(D // 16):
            v = rows_v[r, pl.ds(dg * 16, 16)] + pev[dg]
            plsc.store_scatter(trans_v, [dvecs[dg], bvec], v)

    bufs = ((rows0, tr0, gs0, ws0), (rows1, tr1, gs1, ws1),
            (rows2, tr2, gs2, ws2), (rows3, tr3, gs3, ws3))

    def do_unit(l, b):
      rv, tv, gs, ws = bufs[b]
      wait_gather(rv, gs)

      @pl.when(l >= NBUF)
      def _():
        wait_write(rv, ws)

      transpose_add(l, rv, tv)
      fire_write(l, tv, rv, ws)

      @pl.when(l + NBUF < L)
      def _():
        fire_gather(l + NBUF, rv, gs)

    for b in range(NBUF):
      fire_gather(b, *bufs[b][:1], bufs[b][2])

    def body(k, carry):
      l0 = k * NBUF
      for b in range(NBUF):
        do_unit(l0 + b, b)
      return carry

    lax.fori_loop(0, L // NBUF, body, 0)

    for b in range(NBUF):
      rv, _, _, ws = bufs[b]
      wait_write(rv, ws)

  return emb_kernel


_emb_kernel = _make_kernel()


@jax.jit
def kernel(sequence, token_table, pe):
  seq_t = sequence.T.astype(jnp.int32)
  out5 = _emb_kernel(seq_t, token_table, pe)
  return out5.transpose(2, 4, 0, 1, 3).reshape(B, L, D)


# speedup vs baseline: 1.0313x; 1.0313x over previous
import functools

import jax
import jax.numpy as jnp
from jax import lax
from jax.experimental import pallas as pl
from jax.experimental.pallas import tpu as pltpu
from jax.experimental.pallas import tpu_sc as plsc

B, L, D = 4096, 200, 64
NC, NS = 2, 16
NW = NC * NS
TCB = 128
NTC = B // TCB
NTR = D // 8
NBUF = 3
PITCH = 133


def _make_kernel():
  mesh = plsc.VectorSubcoreMesh(core_axis_name="c", subcore_axis_name="s")

  @functools.partial(
      pl.kernel,
      mesh=mesh,
      compiler_params=pltpu.CompilerParams(use_tc_tiling_on_sc=False,
                                           needs_layout_passes=False),
      out_type=jax.ShapeDtypeStruct((L, NTR, NTC, 8, TCB), jnp.float32),
      scratch_types=(
          [pltpu.VMEM((L, TCB), jnp.int32),
           pltpu.VMEM((L, D), jnp.float32)]
          + [pltpu.VMEM((TCB, D), jnp.float32) for _ in range(NBUF)]
          + [pltpu.VMEM((D, PITCH), jnp.float32) for _ in range(NBUF)]
          + [pltpu.SemaphoreType.DMA for _ in range(2 * NBUF)]
      ),
  )
  def emb_kernel(seq_hbm, table_hbm, pe_hbm, out_hbm,
                 idx_all, pe_v, rows0, rows1, rows2, tr0, tr1, tr2,
                 gs0, gs1, gs2, ws0, ws1, ws2):
    wid = lax.axis_index("s") * NC + lax.axis_index("c")
    b0 = pl.multiple_of(wid * TCB, TCB)

    pltpu.sync_copy(seq_hbm.at[:, pl.ds(b0, TCB)], idx_all)
    pltpu.sync_copy(pe_hbm.at[pl.ds(0, L)], pe_v)

    def fire_gather(l, rows_v, sem):
      pltpu.async_copy(table_hbm.at[idx_all.at[l]], rows_v, sem)

    def wait_gather(rows_v, sem):
      pltpu.make_async_copy(table_hbm.at[pl.ds(0, TCB)], rows_v, sem).wait()

    def fire_write(l, trans_v, rows_v, sem):
      for tr in range(NTR):
        pltpu.async_copy(trans_v.at[pl.ds(tr * 8, 8), pl.ds(0, TCB)],
                         out_hbm.at[l, tr, wid], sem)

    def wait_write(rows_v, sem):
      pltpu.make_async_copy(table_hbm.at[pl.ds(0, TCB)], rows_v, sem).wait()

    iota16 = lax.iota(jnp.int32, 16)
    zeros16 = jnp.zeros((16,), jnp.int32)
    dvecs = [iota16 + (dg * 16) for dg in range(D // 16)]

    def transpose_add(l, rows_v, trans_v):
      pev = [pe_v[l, pl.ds(dg * 16, 16)] for dg in range(D // 16)]

      @plsc.parallel_loop(0, TCB, step=2, unroll=4)
      def rbody(r0):
        for ru in range(2):
          r = r0 + ru
          bvec = zeros16 + r
          for dg in range(D // 16):
            v = rows_v[r, pl.ds(dg * 16, 16)] + pev[dg]
            plsc.store_scatter(trans_v, [dvecs[dg], bvec], v)

    bufs = ((rows0, tr0, gs0, ws0), (rows1, tr1, gs1, ws1),
            (rows2, tr2, gs2, ws2))

    def do_unit(l, b):
      rv, tv, gs, ws = bufs[b]
      wait_gather(rv, gs)

      @pl.when(l >= NBUF)
      def _():
        wait_write(rv, ws)

      transpose_add(l, rv, tv)
      fire_write(l, tv, rv, ws)

      @pl.when(l + NBUF < L)
      def _():
        fire_gather(l + NBUF, rv, gs)

    for b in range(NBUF):
      fire_gather(b, *bufs[b][:1], bufs[b][2])

    def body(k, carry):
      l0 = k * NBUF
      for b in range(NBUF):
        l = l0 + b

        @pl.when(l < L)
        def _():
          do_unit(l, b)

      return carry

    lax.fori_loop(0, (L + NBUF - 1) // NBUF, body, 0)

    for b in range(NBUF):
      rv, _, _, ws = bufs[b]
      wait_write(rv, ws)

  return emb_kernel


_emb_kernel = _make_kernel()

V = 1000000
UC = 256
NFC = V // UC
UTAIL = V - NFC * UC


def _make_untile():
  mesh = plsc.VectorSubcoreMesh(core_axis_name="c", subcore_axis_name="s")

  @functools.partial(
      pl.kernel,
      mesh=mesh,
      compiler_params=pltpu.CompilerParams(use_tc_tiling_on_sc=True,
                                           needs_layout_passes=False),
      out_type=jax.ShapeDtypeStruct((V * D,), jnp.float32),
      scratch_types=(
          [pltpu.VMEM((UC, D), jnp.float32) for _ in range(2)]
          + [pltpu.VMEM((UC * D,), jnp.float32) for _ in range(2)]
          + [pltpu.SemaphoreType.DMA for _ in range(4)]
      ),
  )
  def untile_kernel(table_hbm, out_hbm, in0, in1, fl0, fl1,
                    gs0, gs1, ws0, ws1):
    wid = lax.axis_index("s") * NC + lax.axis_index("c")

    def fire_read(c, in_v, sem):
      off = pl.multiple_of(c * UC, 8)
      pltpu.async_copy(table_hbm.at[pl.ds(off, UC)], in_v, sem)

    def wait_read(in_v, sem):
      pltpu.make_async_copy(table_hbm.at[pl.ds(0, UC)], in_v, sem).wait()

    def fire_write(c, fl_v, sem):
      pltpu.async_copy(fl_v, out_hbm.at[pl.ds(c * (UC * D), UC * D)], sem)

    def wait_write(fl_v, sem):
      pltpu.make_async_copy(out_hbm.at[pl.ds(0, UC * D)], fl_v, sem).wait()

    def compact(in_v, fl_v, nrows):
      @plsc.parallel_loop(0, nrows, step=2, unroll=4)
      def rbody(r0):
        for ru in range(2):
          r = r0 + ru
          for dg in range(D // 16):
            fl_v[pl.ds(r * D + dg * 16, 16)] = in_v[r, pl.ds(dg * 16, 16)]

    bufs = ((in0, fl0, gs0, ws0), (in1, fl1, gs1, ws1))
    fire_read(wid, in0, gs0)
    fire_read(wid + NW, in1, gs1)

    def body(k, carry):
      for b in range(2):
        c = wid + (k * 2 + b) * NW
        iv, fv, gs, ws = bufs[b]

        @pl.when(c < NFC)
        def _():
          wait_read(iv, gs)

          @pl.when(c >= 2 * NW)
          def _():
            wait_write(fv, ws)

          compact(iv, fv, UC)
          fire_write(c, fv, ws)

          @pl.when(c + 2 * NW < NFC)
          def _():
            fire_read(c + 2 * NW, iv, gs)

      return carry

    lax.fori_loop(0, 62, body, 0)
    for b in range(2):
      _, fv, _, ws = bufs[b]
      wait_write(fv, ws)

    @pl.when(wid == NW - 1)
    def _():
      toff = pl.multiple_of(NFC * UC, 8)
      pltpu.sync_copy(table_hbm.at[pl.ds(toff, UTAIL)],
                      in0.at[pl.ds(0, UTAIL)])
      compact(in0, fl0, UTAIL)
      pltpu.sync_copy(fl0.at[pl.ds(0, UTAIL * D)],
                      out_hbm.at[pl.ds(NFC * UC * D, UTAIL * D)])

  return untile_kernel


_untile_kernel = _make_untile()


@jax.jit
def kernel(sequence, token_table, pe):
  seq_t = sequence.T.astype(jnp.int32)
  tbl_lin = _untile_kernel(token_table).reshape(V, D)
  out5 = _emb_kernel(seq_t, tbl_lin, pe)
  return out5.transpose(2, 4, 0, 1, 3).reshape(B, L, D)


# speedup vs baseline: 2.1431x; 2.0781x over previous
import functools

import jax
import jax.numpy as jnp
from jax import lax
from jax.experimental import pallas as pl
from jax.experimental.pallas import tpu as pltpu
from jax.experimental.pallas import tpu_sc as plsc

B, L, D = 4096, 200, 64
NC, NS = 2, 16
NW = NC * NS
TCB = 128
NTC = B // TCB
NTR = D // 8
NBUF = 3
PITCH = 133


def _make_kernel():
  mesh = plsc.VectorSubcoreMesh(core_axis_name="c", subcore_axis_name="s")

  @functools.partial(
      pl.kernel,
      mesh=mesh,
      compiler_params=pltpu.CompilerParams(use_tc_tiling_on_sc=False,
                                           needs_layout_passes=False),
      out_type=jax.ShapeDtypeStruct((L, NTR, NTC, 8, TCB), jnp.float32),
      scratch_types=(
          [pltpu.VMEM((L, TCB), jnp.int32),
           pltpu.VMEM((L, D), jnp.float32)]
          + [pltpu.VMEM((TCB, D), jnp.float32) for _ in range(NBUF)]
          + [pltpu.VMEM((D, PITCH), jnp.float32) for _ in range(NBUF)]
          + [pltpu.SemaphoreType.DMA for _ in range(2 * NBUF)]
      ),
  )
  def emb_kernel(seq_hbm, table_hbm, pe_hbm, out_hbm,
                 idx_all, pe_v, rows0, rows1, rows2, tr0, tr1, tr2,
                 gs0, gs1, gs2, ws0, ws1, ws2):
    wid = lax.axis_index("s") * NC + lax.axis_index("c")
    b0 = pl.multiple_of(wid * TCB, TCB)

    pltpu.sync_copy(seq_hbm.at[:, pl.ds(b0, TCB)], idx_all)
    pltpu.sync_copy(pe_hbm.at[pl.ds(0, L)], pe_v)

    def fire_gather(l, rows_v, sem):
      pltpu.async_copy(table_hbm.at[idx_all.at[l]], rows_v, sem)

    def wait_gather(rows_v, sem):
      pltpu.make_async_copy(table_hbm.at[pl.ds(0, TCB)], rows_v, sem).wait()

    def fire_write(l, trans_v, rows_v, sem):
      for tr in range(NTR):
        pltpu.async_copy(trans_v.at[pl.ds(tr * 8, 8), pl.ds(0, TCB)],
                         out_hbm.at[l, tr, wid], sem)

    def wait_write(rows_v, sem):
      pltpu.make_async_copy(table_hbm.at[pl.ds(0, TCB)], rows_v, sem).wait()

    iota16 = lax.iota(jnp.int32, 16)
    zeros16 = jnp.zeros((16,), jnp.int32)
    dvecs = [iota16 + (dg * 16) for dg in range(D // 16)]

    def transpose_add(l, rows_v, trans_v):
      pev = [pe_v[l, pl.ds(dg * 16, 16)] for dg in range(D // 16)]

      @plsc.parallel_loop(0, TCB, step=2, unroll=4)
      def rbody(r0):
        for ru in range(2):
          r = r0 + ru
          bvec = zeros16 + r
          for dg in range(D // 16):
            v = rows_v[r, pl.ds(dg * 16, 16)] + pev[dg]
            plsc.store_scatter(trans_v, [dvecs[dg], bvec], v)

    bufs = ((rows0, tr0, gs0, ws0), (rows1, tr1, gs1, ws1),
            (rows2, tr2, gs2, ws2))

    def do_unit(l, b):
      rv, tv, gs, ws = bufs[b]
      wait_gather(rv, gs)

      @pl.when(l >= NBUF)
      def _():
        wait_write(rv, ws)

      transpose_add(l, rv, tv)
      fire_write(l, tv, rv, ws)

      @pl.when(l + NBUF < L)
      def _():
        fire_gather(l + NBUF, rv, gs)

    for b in range(NBUF):
      fire_gather(b, *bufs[b][:1], bufs[b][2])

    def body(k, carry):
      l0 = k * NBUF
      for b in range(NBUF):
        l = l0 + b

        @pl.when(l < L)
        def _():
          do_unit(l, b)

      return carry

    lax.fori_loop(0, (L + NBUF - 1) // NBUF, body, 0)

    for b in range(NBUF):
      rv, _, _, ws = bufs[b]
      wait_write(rv, ws)

  return emb_kernel


_emb_kernel = _make_kernel()

V = 1000000
UC = 256
NFC = V // UC
UTAIL = V - NFC * UC
UPITCH = 65


def _make_table_transpose():
  mesh = plsc.VectorSubcoreMesh(core_axis_name="c", subcore_axis_name="s")

  @functools.partial(
      pl.kernel,
      mesh=mesh,
      compiler_params=pltpu.CompilerParams(use_tc_tiling_on_sc=True,
                                           needs_layout_passes=False),
      out_type=jax.ShapeDtypeStruct((V * D,), jnp.float32),
      scratch_types=(
          [pltpu.VMEM((D, UC), jnp.float32) for _ in range(2)]
          + [pltpu.VMEM((UC * UPITCH,), jnp.float32)]
          + [pltpu.VMEM((UC * D,), jnp.float32) for _ in range(2)]
          + [pltpu.SemaphoreType.DMA for _ in range(4)]
      ),
  )
  def ttr_kernel(tablet_hbm, tail_hbm, out_hbm, in0, in1, stage, fl0, fl1,
                 gs0, gs1, ws0, ws1):
    wid = lax.axis_index("s") * NC + lax.axis_index("c")

    iota16 = lax.iota(jnp.int32, 16)
    civecs = [(iota16 + cg * 16) * UPITCH for cg in range(UC // 16)]

    def fire_read(c, in_v, sem):
      off = pl.multiple_of(c * UC, 128)
      pltpu.async_copy(tablet_hbm.at[:, pl.ds(off, UC)], in_v, sem)

    def wait_read(in_v, sem):
      pltpu.make_async_copy(tablet_hbm.at[:, pl.ds(0, UC)], in_v, sem).wait()

    def fire_write(c, fl_v, sem):
      pltpu.async_copy(fl_v, out_hbm.at[pl.ds(c * (UC * D), UC * D)], sem)

    def wait_write(fl_v, sem):
      pltpu.make_async_copy(out_hbm.at[pl.ds(0, UC * D)], fl_v, sem).wait()

    def transpose_chunk(in_v, fl_v, ncols):
      @plsc.parallel_loop(0, D, step=2, unroll=4)
      def dbody(d0):
        for du in range(2):
          d = d0 + du
          for cg in range(ncols // 16):
            plsc.store_scatter(stage, [civecs[cg] + d],
                               in_v[d, pl.ds(cg * 16, 16)])

      @plsc.parallel_loop(0, ncols, step=2, unroll=4)
      def cbody(c0):
        for cu in range(2):
          c = c0 + cu
          for dg in range(D // 16):
            fl_v[pl.ds(c * D + dg * 16, 16)] = (
                stage[pl.ds(c * UPITCH + dg * 16, 16)])

    bufs = ((in0, fl0, gs0, ws0), (in1, fl1, gs1, ws1))
    fire_read(wid, in0, gs0)
    fire_read(wid + NW, in1, gs1)

    def body(k, carry):
      for b in range(2):
        c = wid + (k * 2 + b) * NW
        iv, fv, gs, ws = bufs[b]

        @pl.when(c < NFC)
        def _():
          wait_read(iv, gs)

          @pl.when(c >= 2 * NW)
          def _():
            wait_write(fv, ws)

          transpose_chunk(iv, fv, UC)
          fire_write(c, fv, ws)

          @pl.when(c + 2 * NW < NFC)
          def _():
            fire_read(c + 2 * NW, iv, gs)

      return carry

    lax.fori_loop(0, 62, body, 0)
    for b in range(2):
      _, fv, _, ws = bufs[b]
      wait_write(fv, ws)

    @pl.when(wid == NW - 1)
    def _():
      pltpu.sync_copy(tail_hbm, in0.at[:, pl.ds(0, 128)])
      transpose_chunk(in0, fl0, UTAIL)
      pltpu.sync_copy(fl0.at[pl.ds(0, UTAIL * D)],
                      out_hbm.at[pl.ds(NFC * UC * D, UTAIL * D)])

  return ttr_kernel


_ttr_kernel = _make_table_transpose()


@jax.jit
def kernel(sequence, token_table, pe):
  seq_t = sequence.T.astype(jnp.int32)
  table_t = token_table.T
  tail = jnp.pad(table_t[:, NFC * UC:], ((0, 0), (0, 128 - UTAIL)))
  tbl_lin = _ttr_kernel(table_t, tail).reshape(V, D)
  out5 = _emb_kernel(seq_t, tbl_lin, pe)
  return out5.transpose(2, 4, 0, 1, 3).reshape(B, L, D)
